# Initial kernel scaffold; baseline (speedup 1.0000x reference)
import jax
import jax.numpy as jnp
from jax.experimental import pallas as pl


def kernel(x, f, s, W_e1, b_e1, W_e2, b_e2, W_d1, b_d1, W_d2, b_d2):
    raise NotImplementedError("write your pallas kernel here")


# speedup vs baseline: 20.7271x; 20.7271x over previous
import functools

import jax
import jax.numpy as jnp
from jax import lax
from jax.experimental import pallas as pl
from jax.experimental.pallas import tpu as pltpu
from jax.experimental.pallas import tpu_sc as plsc

_N = 10000
_E = 320000
_DIM = 128
_HID = 64
_ALPHA = 1.0
_BETA = 0.0
_GAMMA = 0.001

_NC = 2
_NS = 16
_NW = _NC * _NS
_EPT = _E // _NW
_CH = 80
_NCH = _EPT // _CH
_NPAD = 10240
_ZSEG = 640
_RPT = _NPAD // _NS
_ZROWS = 128

_BR = 1000
_GB = _N // _BR

_mesh = plsc.VectorSubcoreMesh(core_axis_name="c", subcore_axis_name="s")


@functools.partial(
    pl.kernel,
    out_type=jax.ShapeDtypeStruct((_NC * 2 * _NPAD,), jnp.float32),
    mesh=_mesh,
    compiler_params=pltpu.CompilerParams(use_tc_tiling_on_sc=False),
    scratch_types=[
        pltpu.VMEM((_NCH, _CH), jnp.int32),
        pltpu.VMEM((_NCH, _CH), jnp.int32),
        pltpu.VMEM((_CH,), jnp.float32),
        pltpu.VMEM((_ZSEG,), jnp.float32),
        pltpu.VMEM_SHARED((_NPAD,), jnp.float32),
        pltpu.VMEM_SHARED((_NPAD,), jnp.float32),
        pltpu.SemaphoreType.DMA,
    ],
)
def _deg_kernel(src_hbm, dst_hbm, out_hbm, src_v, dst_v, ones_v, zb_v,
                din_sh, dout_sh, sem):
    c = lax.axis_index("c")
    s = lax.axis_index("s")
    w = c * _NS + s
    one16 = jnp.full((16,), 1.0, jnp.float32)
    zero16 = jnp.zeros((16,), jnp.float32)
    for l in range(_CH // 16):
        ones_v[pl.ds(l * 16, 16)] = one16

    def _zf(i, _):
        zb_v[pl.ds(i * 16, 16)] = zero16
        return None

    lax.fori_loop(0, _ZSEG // 16, _zf, None)
    pltpu.sync_copy(zb_v, din_sh.at[pl.ds(s * _ZSEG, _ZSEG)])
    pltpu.sync_copy(zb_v, dout_sh.at[pl.ds(s * _ZSEG, _ZSEG)])
    pltpu.sync_copy(src_hbm.at[w], src_v)
    pltpu.sync_copy(dst_hbm.at[w], dst_v)
    plsc.subcore_barrier()

    def _body(j, _):
        pltpu.async_copy(ones_v, dout_sh.at[src_v.at[j]], sem, add=True)
        pltpu.async_copy(ones_v, din_sh.at[dst_v.at[j]], sem, add=True)

        @pl.when(j >= 8)
        def _drain():
            pltpu.make_async_copy(ones_v, dout_sh.at[src_v.at[0]], sem).wait()
            pltpu.make_async_copy(ones_v, din_sh.at[dst_v.at[0]], sem).wait()

        return None

    lax.fori_loop(0, _NCH, _body, None)

    def _dr(j, _):
        pltpu.make_async_copy(ones_v, dout_sh.at[src_v.at[0]], sem).wait()
        pltpu.make_async_copy(ones_v, din_sh.at[dst_v.at[0]], sem).wait()
        return None

    lax.fori_loop(0, 8, _dr, None)
    plsc.subcore_barrier()
    pltpu.sync_copy(din_sh.at[pl.ds(s * _ZSEG, _ZSEG)],
                    out_hbm.at[pl.ds(c * 2 * _NPAD + s * _ZSEG, _ZSEG)])
    pltpu.sync_copy(dout_sh.at[pl.ds(s * _ZSEG, _ZSEG)],
                    out_hbm.at[pl.ds(c * 2 * _NPAD + _NPAD + s * _ZSEG, _ZSEG)])


def _make_prop(d):
    @functools.partial(
        pl.kernel,
        out_type=jax.ShapeDtypeStruct((_NC, _NPAD, d), jnp.float32),
        mesh=_mesh,
        compiler_params=pltpu.CompilerParams(use_tc_tiling_on_sc=False),
        scratch_types=[
            pltpu.VMEM((_NCH, _CH), jnp.int32),
            pltpu.VMEM((_NCH, _CH), jnp.int32),
            pltpu.VMEM((_CH, d), jnp.float32),
            pltpu.VMEM((_CH, d), jnp.float32),
            pltpu.VMEM((_ZROWS, d), jnp.float32),
            pltpu.VMEM_SHARED((_NPAD, d), jnp.float32),
            pltpu.SemaphoreType.DMA,
            pltpu.SemaphoreType.DMA,
        ],
    )
    def _prop(g_hbm, src_hbm, dst_hbm, out_hbm, src_v, dst_v, buf_a, buf_b,
              zb, acc, sem_a, sem_b):
        c = lax.axis_index("c")
        s = lax.axis_index("s")
        w = c * _NS + s
        zero16 = jnp.zeros((16,), jnp.float32)

        def _zf(i, _):
            for l in range(d // 16):
                zb[i, pl.ds(l * 16, 16)] = zero16
            return None

        lax.fori_loop(0, _ZROWS, _zf, None)
        pltpu.sync_copy(src_hbm.at[w], src_v)
        pltpu.sync_copy(dst_hbm.at[w], dst_v)
        for z in range(_RPT // _ZROWS):
            pltpu.sync_copy(zb, acc.at[pl.ds(s * _RPT + z * _ZROWS, _ZROWS)])
        gsrc = g_hbm
        plsc.subcore_barrier()

        pltpu.async_copy(gsrc.at[src_v.at[0]], buf_a, sem_a)

        def _body(j2, _):
            j = j2 * 2
            pltpu.async_copy(gsrc.at[src_v.at[j + 1]], buf_b, sem_b)
            pltpu.make_async_copy(gsrc.at[src_v.at[j]], buf_a, sem_a).wait()
            pltpu.sync_copy(buf_a, acc.at[dst_v.at[j]], add=True)
            pltpu.async_copy(gsrc.at[src_v.at[j + 2]], buf_a, sem_a)
            pltpu.make_async_copy(gsrc.at[src_v.at[j + 1]], buf_b, sem_b).wait()
            pltpu.sync_copy(buf_b, acc.at[dst_v.at[j + 1]], add=True)
            return None

        lax.fori_loop(0, (_NCH - 1) // 2, _body, None)
        pltpu.make_async_copy(gsrc.at[src_v.at[_NCH - 1]], buf_a, sem_a).wait()
        pltpu.sync_copy(buf_a, acc.at[dst_v.at[_NCH - 1]], add=True)
        plsc.subcore_barrier()
        pltpu.sync_copy(acc.at[pl.ds(s * _RPT, _RPT)],
                        out_hbm.at[c, pl.ds(s * _RPT, _RPT)])

    return _prop


_prop64 = _make_prop(_HID)


def _stage_a_body(x_ref, dp_ref, w_ref, g1_ref, dinv_ref, dn_ref):
    dp = dp_ref[...]
    din = dp[:, 0:1] + dp[:, 2:3]
    dout = dp[:, 1:2] + dp[:, 3:4]
    dinv = lax.rsqrt(jnp.maximum(din + 1.0, 1e-12))
    dinv_ref[...] = dinv
    dn_ref[...] = din + dout
    g1_ref[...] = dinv * jnp.dot(x_ref[...], w_ref[...],
                                 preferred_element_type=jnp.float32)


_stage_a = pl.pallas_call(
    _stage_a_body,
    grid=(_GB,),
    in_specs=[
        pl.BlockSpec((_BR, _DIM), lambda i: (i, 0)),
        pl.BlockSpec((_BR, 4), lambda i: (i, 0)),
        pl.BlockSpec((_DIM, _HID), lambda i: (0, 0)),
    ],
    out_specs=[
        pl.BlockSpec((_BR, _HID), lambda i: (i, 0)),
        pl.BlockSpec((_BR, 1), lambda i: (i, 0)),
        pl.BlockSpec((_BR, 1), lambda i: (i, 0)),
    ],
    out_shape=[
        jax.ShapeDtypeStruct((_N, _HID), jnp.float32),
        jax.ShapeDtypeStruct((_N, 1), jnp.float32),
        jax.ShapeDtypeStruct((_N, 1), jnp.float32),
    ],
)


def _mid_body(relu, t_ref, g_ref, dinv_ref, b_ref, w_ref, out_ref):
    t = t_ref[0] + t_ref[1]
    pre = dinv_ref[...] * (t + g_ref[...]) + b_ref[...]
    h = jnp.maximum(pre, 0.0) if relu else pre
    out_ref[...] = dinv_ref[...] * jnp.dot(h, w_ref[...],
                                           preferred_element_type=jnp.float32)


def _make_mid(relu, dout):
    return pl.pallas_call(
        functools.partial(_mid_body, relu),
        grid=(_GB,),
        in_specs=[
            pl.BlockSpec((_NC, _BR, _HID), lambda i: (0, i, 0)),
            pl.BlockSpec((_BR, _HID), lambda i: (i, 0)),
            pl.BlockSpec((_BR, 1), lambda i: (i, 0)),
            pl.BlockSpec((1, _HID), lambda i: (0, 0)),
            pl.BlockSpec((_HID, dout), lambda i: (0, 0)),
        ],
        out_specs=pl.BlockSpec((_BR, dout), lambda i: (i, 0)),
        out_shape=jax.ShapeDtypeStruct((_N, dout), jnp.float32),
    )


_stage_b = _make_mid(True, _HID)
_stage_c = _make_mid(False, _HID)


def _stage_d_body(t_ref, g_ref, dinv_ref, b_ref, out_ref):
    t = t_ref[0] + t_ref[1]
    pre = dinv_ref[...] * (t + g_ref[...]) + b_ref[...]
    out_ref[...] = dinv_ref[...] * jnp.maximum(pre, 0.0)


_stage_d = pl.pallas_call(
    _stage_d_body,
    grid=(_GB,),
    in_specs=[
        pl.BlockSpec((_NC, _BR, _HID), lambda i: (0, i, 0)),
        pl.BlockSpec((_BR, _HID), lambda i: (i, 0)),
        pl.BlockSpec((_BR, 1), lambda i: (i, 0)),
        pl.BlockSpec((1, _HID), lambda i: (0, 0)),
    ],
    out_specs=pl.BlockSpec((_BR, _HID), lambda i: (i, 0)),
    out_shape=jax.ShapeDtypeStruct((_N, _HID), jnp.float32),
)


def _stage_e_body(t_ref, g_ref, dinv_ref, w_ref, b_ref, x_ref, out_ref):
    t = t_ref[0] + t_ref[1]
    s4 = dinv_ref[...] * (t + g_ref[...])
    dec = jnp.dot(s4, w_ref[...], preferred_element_type=jnp.float32) + b_ref[...]
    out_ref[...] = _BETA * x_ref[...] + (1.0 - _BETA) * dec


_stage_e = pl.pallas_call(
    _stage_e_body,
    grid=(_GB,),
    in_specs=[
        pl.BlockSpec((_NC, _BR, _HID), lambda i: (0, i, 0)),
        pl.BlockSpec((_BR, _HID), lambda i: (i, 0)),
        pl.BlockSpec((_BR, 1), lambda i: (i, 0)),
        pl.BlockSpec((_HID, _DIM), lambda i: (0, 0)),
        pl.BlockSpec((1, _DIM), lambda i: (0, 0)),
        pl.BlockSpec((_BR, _DIM), lambda i: (i, 0)),
    ],
    out_specs=pl.BlockSpec((_BR, _DIM), lambda i: (i, 0)),
    out_shape=jax.ShapeDtypeStruct((_N, _DIM), jnp.float32),
)


def _stage_f_body(x_ref, xn_ref, t5a_ref, t5b_ref, dn_ref, out_ref, acc):
    i = pl.program_id(0)

    @pl.when(i == 0)
    def _init():
        acc[0] = 0.0
        acc[1] = 0.0
        acc[2] = 0.0

    xn = xn_ref[...]
    t5a = t5a_ref[0] + t5a_ref[1]
    t5b = t5b_ref[0] + t5b_ref[1]
    cross = (jnp.sum(xn[:, :_HID] * t5a) + jnp.sum(xn[:, _HID:] * t5b))
    nrow = jnp.sum(xn * xn, axis=1, keepdims=True)
    esq = jnp.sum(nrow * dn_ref[...])
    dd = x_ref[...] - xn
    scs = jnp.sum(jnp.sqrt(jnp.sum(dd * dd, axis=1, keepdims=True) + 1e-12))
    acc[0] += cross
    acc[1] += esq
    acc[2] += scs

    @pl.when(i == _GB - 1)
    def _fin():
        out_ref[0, 0] = (_ALPHA * (acc[2] / _N)
                         + _GAMMA * ((acc[1] - 2.0 * acc[0]) / _E))


_stage_f = pl.pallas_call(
    _stage_f_body,
    grid=(_GB,),
    in_specs=[
        pl.BlockSpec((_BR, _DIM), lambda i: (i, 0)),
        pl.BlockSpec((_BR, _DIM), lambda i: (i, 0)),
        pl.BlockSpec((_NC, _BR, _HID), lambda i: (0, i, 0)),
        pl.BlockSpec((_NC, _BR, _HID), lambda i: (0, i, 0)),
        pl.BlockSpec((_BR, 1), lambda i: (i, 0)),
    ],
    out_specs=pl.BlockSpec(memory_space=pltpu.SMEM),
    out_shape=jax.ShapeDtypeStruct((1, 1), jnp.float32),
    scratch_shapes=[pltpu.SMEM((4,), jnp.float32)],
)


def kernel(x, f, s, W_e1, b_e1, W_e2, b_e2, W_d1, b_d1, W_d2, b_d2):
    src = s[0].reshape(_NW, _NCH, _CH)
    dst = s[1].reshape(_NW, _NCH, _CH)

    degp = _deg_kernel(src, dst).reshape(_NC, 2, _NPAD)
    degp2 = degp[:, :, :_N].transpose(2, 0, 1).reshape(_N, 4)

    g1, dinv, dn = _stage_a(x, degp2, W_e1)
    t1 = _prop64(g1, src, dst)[:, :_N]
    g2 = _stage_b(t1, g1, dinv, b_e1.reshape(1, _HID), W_e2)
    t2 = _prop64(g2, src, dst)[:, :_N]
    g3 = _stage_c(t2, g2, dinv, b_e2.reshape(1, _HID), W_d1)
    t3 = _prop64(g3, src, dst)[:, :_N]
    g4 = _stage_d(t3, g3, dinv, b_d1.reshape(1, _HID))
    t4 = _prop64(g4, src, dst)[:, :_N]
    x_new = _stage_e(t4, g4, dinv, W_d2, b_d2.reshape(1, _DIM), x)
    t5a = _prop64(x_new[:, :_HID], src, dst)[:, :_N]
    t5b = _prop64(x_new[:, _HID:], src, dst)[:, :_N]
    lossm = _stage_f(x, x_new, t5a, t5b, dn)
    return (lossm[0, 0], x_new)


# speedup vs baseline: 25.6754x; 1.2387x over previous
import functools

import jax
import jax.numpy as jnp
from jax import lax
from jax.experimental import pallas as pl
from jax.experimental.pallas import tpu as pltpu
from jax.experimental.pallas import tpu_sc as plsc

_N = 10000
_E = 320000
_DIM = 128
_HID = 64
_ALPHA = 1.0
_BETA = 0.0
_GAMMA = 0.001

_NC = 2
_NS = 16
_NW = _NC * _NS
_EPT = _E // _NW
_CH = 80
_NCH = _EPT // _CH
_NPAD = 10240
_ZSEG = 640
_RPT = _NPAD // _NS
_ZROWS = 128

_BR = 1000
_GB = _N // _BR

_mesh = plsc.VectorSubcoreMesh(core_axis_name="c", subcore_axis_name="s")


@functools.partial(
    pl.kernel,
    out_type=jax.ShapeDtypeStruct((_NC * 2 * _NPAD,), jnp.float32),
    mesh=_mesh,
    compiler_params=pltpu.CompilerParams(use_tc_tiling_on_sc=False),
    scratch_types=[
        pltpu.VMEM((_NCH, _CH), jnp.int32),
        pltpu.VMEM((_NCH, _CH), jnp.int32),
        pltpu.VMEM((_CH,), jnp.float32),
        pltpu.VMEM((_ZSEG,), jnp.float32),
        pltpu.VMEM_SHARED((_NPAD,), jnp.float32),
        pltpu.VMEM_SHARED((_NPAD,), jnp.float32),
        pltpu.SemaphoreType.DMA,
    ],
)
def _deg_kernel(src_hbm, dst_hbm, out_hbm, src_v, dst_v, ones_v, zb_v,
                din_sh, dout_sh, sem):
    c = lax.axis_index("c")
    s = lax.axis_index("s")
    w = c * _NS + s
    one16 = jnp.full((16,), 1.0, jnp.float32)
    zero16 = jnp.zeros((16,), jnp.float32)
    for l in range(_CH // 16):
        ones_v[pl.ds(l * 16, 16)] = one16

    def _zf(i, _):
        zb_v[pl.ds(i * 16, 16)] = zero16
        return None

    lax.fori_loop(0, _ZSEG // 16, _zf, None)
    pltpu.sync_copy(zb_v, din_sh.at[pl.ds(s * _ZSEG, _ZSEG)])
    pltpu.sync_copy(zb_v, dout_sh.at[pl.ds(s * _ZSEG, _ZSEG)])
    pltpu.sync_copy(src_hbm.at[w], src_v)
    pltpu.sync_copy(dst_hbm.at[w], dst_v)
    plsc.subcore_barrier()

    def _body(j, _):
        pltpu.async_copy(ones_v, dout_sh.at[src_v.at[j]], sem, add=True)
        pltpu.async_copy(ones_v, din_sh.at[dst_v.at[j]], sem, add=True)

        @pl.when(j >= 8)
        def _drain():
            pltpu.make_async_copy(ones_v, dout_sh.at[src_v.at[0]], sem).wait()
            pltpu.make_async_copy(ones_v, din_sh.at[dst_v.at[0]], sem).wait()

        return None

    lax.fori_loop(0, _NCH, _body, None)

    def _dr(j, _):
        pltpu.make_async_copy(ones_v, dout_sh.at[src_v.at[0]], sem).wait()
        pltpu.make_async_copy(ones_v, din_sh.at[dst_v.at[0]], sem).wait()
        return None

    lax.fori_loop(0, 8, _dr, None)
    plsc.subcore_barrier()
    pltpu.sync_copy(din_sh.at[pl.ds(s * _ZSEG, _ZSEG)],
                    out_hbm.at[pl.ds(c * 2 * _NPAD + s * _ZSEG, _ZSEG)])
    pltpu.sync_copy(dout_sh.at[pl.ds(s * _ZSEG, _ZSEG)],
                    out_hbm.at[pl.ds(c * 2 * _NPAD + _NPAD + s * _ZSEG, _ZSEG)])


def _make_prop(d):
    @functools.partial(
        pl.kernel,
        out_type=jax.ShapeDtypeStruct((_NC, _NPAD, d), jnp.float32),
        mesh=_mesh,
        compiler_params=pltpu.CompilerParams(use_tc_tiling_on_sc=False),
        scratch_types=[
            pltpu.VMEM((_NCH, _CH), jnp.int32),
            pltpu.VMEM((_NCH, _CH), jnp.int32),
            pltpu.VMEM((_CH, d), jnp.float32),
            pltpu.VMEM((_CH, d), jnp.float32),
            pltpu.VMEM((_CH, d), jnp.float32),
            pltpu.VMEM((_CH, d), jnp.float32),
            pltpu.VMEM((_ZROWS, d), jnp.float32),
            pltpu.VMEM_SHARED((_NPAD, d), jnp.float32),
        ] + [pltpu.SemaphoreType.DMA] * 8,
    )
    def _prop(g_hbm, src_hbm, dst_hbm, out_hbm, src_v, dst_v, b0, b1, b2, b3,
              zb, acc, g0, g1, g2, g3, s0, s1, s2, s3):
        c = lax.axis_index("c")
        s = lax.axis_index("s")
        w = c * _NS + s
        bufs = (b0, b1, b2, b3)
        gsem = (g0, g1, g2, g3)
        ssem = (s0, s1, s2, s3)
        zero16 = jnp.zeros((16,), jnp.float32)

        def _zf(i, _):
            for l in range(d // 16):
                zb[i, pl.ds(l * 16, 16)] = zero16
            return None

        lax.fori_loop(0, _ZROWS, _zf, None)
        pltpu.sync_copy(src_hbm.at[w], src_v)
        pltpu.sync_copy(dst_hbm.at[w], dst_v)
        for z in range(_RPT // _ZROWS):
            pltpu.sync_copy(zb, acc.at[pl.ds(s * _RPT + z * _ZROWS, _ZROWS)])
        plsc.subcore_barrier()

        def _fire_g(j, b):
            pltpu.async_copy(g_hbm.at[src_v.at[j]], bufs[b], gsem[b])

        def _wait_g(j, b):
            pltpu.make_async_copy(g_hbm.at[src_v.at[j]], bufs[b], gsem[b]).wait()

        def _fire_s(j, b):
            pltpu.async_copy(bufs[b], acc.at[dst_v.at[j]], ssem[b], add=True)

        def _wait_s(j, b):
            pltpu.make_async_copy(bufs[b], acc.at[dst_v.at[j]], ssem[b]).wait()

        _fire_g(0, 0)
        _fire_g(1, 1)
        _fire_g(2, 2)
        _wait_g(0, 0)
        _fire_s(0, 0)
        _fire_g(3, 3)
        _wait_g(1, 1)
        _fire_s(1, 1)

        def _body(j2, _):
            jb = 2 + j2 * 4
            for u in range(4):
                j = jb + u
                _wait_s(j - 2, u)
                _fire_g(j + 2, u)
                _wait_g(j, (2 + u) % 4)
                _fire_s(j, (2 + u) % 4)
            return None

        lax.fori_loop(0, 30, _body, None)
        _wait_s(120, 0)
        _fire_g(124, 0)
        _wait_g(122, 2)
        _fire_s(122, 2)
        _wait_s(121, 1)
        _wait_g(123, 3)
        _fire_s(123, 3)
        _wait_s(122, 2)
        _wait_g(124, 0)
        _fire_s(124, 0)
        _wait_s(123, 3)
        _wait_s(124, 0)
        plsc.subcore_barrier()
        pltpu.sync_copy(acc.at[pl.ds(s * _RPT, _RPT)],
                        out_hbm.at[c, pl.ds(s * _RPT, _RPT)])

    return _prop


_prop64 = _make_prop(_HID)


def _stage_a_body(x_ref, dp_ref, w_ref, g1_ref, dinv_ref, dn_ref):
    dp = dp_ref[...]
    din = dp[:, 0:1] + dp[:, 2:3]
    dout = dp[:, 1:2] + dp[:, 3:4]
    dinv = lax.rsqrt(jnp.maximum(din + 1.0, 1e-12))
    dinv_ref[...] = dinv
    dn_ref[...] = din + dout
    g1_ref[...] = dinv * jnp.dot(x_ref[...], w_ref[...],
                                 preferred_element_type=jnp.float32)


_stage_a = pl.pallas_call(
    _stage_a_body,
    grid=(_GB,),
    in_specs=[
        pl.BlockSpec((_BR, _DIM), lambda i: (i, 0)),
        pl.BlockSpec((_BR, 4), lambda i: (i, 0)),
        pl.BlockSpec((_DIM, _HID), lambda i: (0, 0)),
    ],
    out_specs=[
        pl.BlockSpec((_BR, _HID), lambda i: (i, 0)),
        pl.BlockSpec((_BR, 1), lambda i: (i, 0)),
        pl.BlockSpec((_BR, 1), lambda i: (i, 0)),
    ],
    out_shape=[
        jax.ShapeDtypeStruct((_N, _HID), jnp.float32),
        jax.ShapeDtypeStruct((_N, 1), jnp.float32),
        jax.ShapeDtypeStruct((_N, 1), jnp.float32),
    ],
)


def _mid_body(relu, t_ref, g_ref, dinv_ref, b_ref, w_ref, out_ref):
    t = t_ref[0] + t_ref[1]
    pre = dinv_ref[...] * (t + g_ref[...]) + b_ref[...]
    h = jnp.maximum(pre, 0.0) if relu else pre
    out_ref[...] = dinv_ref[...] * jnp.dot(h, w_ref[...],
                                           preferred_element_type=jnp.float32)


def _make_mid(relu, dout):
    return pl.pallas_call(
        functools.partial(_mid_body, relu),
        grid=(_GB,),
        in_specs=[
            pl.BlockSpec((_NC, _BR, _HID), lambda i: (0, i, 0)),
            pl.BlockSpec((_BR, _HID), lambda i: (i, 0)),
            pl.BlockSpec((_BR, 1), lambda i: (i, 0)),
            pl.BlockSpec((1, _HID), lambda i: (0, 0)),
            pl.BlockSpec((_HID, dout), lambda i: (0, 0)),
        ],
        out_specs=pl.BlockSpec((_BR, dout), lambda i: (i, 0)),
        out_shape=jax.ShapeDtypeStruct((_N, dout), jnp.float32),
    )


_stage_b = _make_mid(True, _HID)
_stage_c = _make_mid(False, _HID)


def _stage_d_body(t_ref, g_ref, dinv_ref, b_ref, out_ref):
    t = t_ref[0] + t_ref[1]
    pre = dinv_ref[...] * (t + g_ref[...]) + b_ref[...]
    out_ref[...] = dinv_ref[...] * jnp.maximum(pre, 0.0)


_stage_d = pl.pallas_call(
    _stage_d_body,
    grid=(_GB,),
    in_specs=[
        pl.BlockSpec((_NC, _BR, _HID), lambda i: (0, i, 0)),
        pl.BlockSpec((_BR, _HID), lambda i: (i, 0)),
        pl.BlockSpec((_BR, 1), lambda i: (i, 0)),
        pl.BlockSpec((1, _HID), lambda i: (0, 0)),
    ],
    out_specs=pl.BlockSpec((_BR, _HID), lambda i: (i, 0)),
    out_shape=jax.ShapeDtypeStruct((_N, _HID), jnp.float32),
)


def _stage_e_body(t_ref, g_ref, dinv_ref, w_ref, b_ref, x_ref, out_ref):
    t = t_ref[0] + t_ref[1]
    s4 = dinv_ref[...] * (t + g_ref[...])
    dec = jnp.dot(s4, w_ref[...], preferred_element_type=jnp.float32) + b_ref[...]
    out_ref[...] = _BETA * x_ref[...] + (1.0 - _BETA) * dec


_stage_e = pl.pallas_call(
    _stage_e_body,
    grid=(_GB,),
    in_specs=[
        pl.BlockSpec((_NC, _BR, _HID), lambda i: (0, i, 0)),
        pl.BlockSpec((_BR, _HID), lambda i: (i, 0)),
        pl.BlockSpec((_BR, 1), lambda i: (i, 0)),
        pl.BlockSpec((_HID, _DIM), lambda i: (0, 0)),
        pl.BlockSpec((1, _DIM), lambda i: (0, 0)),
        pl.BlockSpec((_BR, _DIM), lambda i: (i, 0)),
    ],
    out_specs=pl.BlockSpec((_BR, _DIM), lambda i: (i, 0)),
    out_shape=jax.ShapeDtypeStruct((_N, _DIM), jnp.float32),
)


def _stage_f_body(x_ref, xn_ref, t5a_ref, t5b_ref, dn_ref, out_ref, acc):
    i = pl.program_id(0)

    @pl.when(i == 0)
    def _init():
        acc[0] = 0.0
        acc[1] = 0.0
        acc[2] = 0.0

    xn = xn_ref[...]
    t5a = t5a_ref[0] + t5a_ref[1]
    t5b = t5b_ref[0] + t5b_ref[1]
    cross = (jnp.sum(xn[:, :_HID] * t5a) + jnp.sum(xn[:, _HID:] * t5b))
    nrow = jnp.sum(xn * xn, axis=1, keepdims=True)
    esq = jnp.sum(nrow * dn_ref[...])
    dd = x_ref[...] - xn
    scs = jnp.sum(jnp.sqrt(jnp.sum(dd * dd, axis=1, keepdims=True) + 1e-12))
    acc[0] += cross
    acc[1] += esq
    acc[2] += scs

    @pl.when(i == _GB - 1)
    def _fin():
        out_ref[0, 0] = (_ALPHA * (acc[2] / _N)
                         + _GAMMA * ((acc[1] - 2.0 * acc[0]) / _E))


_stage_f = pl.pallas_call(
    _stage_f_body,
    grid=(_GB,),
    in_specs=[
        pl.BlockSpec((_BR, _DIM), lambda i: (i, 0)),
        pl.BlockSpec((_BR, _DIM), lambda i: (i, 0)),
        pl.BlockSpec((_NC, _BR, _HID), lambda i: (0, i, 0)),
        pl.BlockSpec((_NC, _BR, _HID), lambda i: (0, i, 0)),
        pl.BlockSpec((_BR, 1), lambda i: (i, 0)),
    ],
    out_specs=pl.BlockSpec(memory_space=pltpu.SMEM),
    out_shape=jax.ShapeDtypeStruct((1, 1), jnp.float32),
    scratch_shapes=[pltpu.SMEM((4,), jnp.float32)],
)


def kernel(x, f, s, W_e1, b_e1, W_e2, b_e2, W_d1, b_d1, W_d2, b_d2):
    src = s[0].reshape(_NW, _NCH, _CH)
    dst = s[1].reshape(_NW, _NCH, _CH)

    degp = _deg_kernel(src, dst).reshape(_NC, 2, _NPAD)
    degp2 = degp[:, :, :_N].transpose(2, 0, 1).reshape(_N, 4)

    g1, dinv, dn = _stage_a(x, degp2, W_e1)
    t1 = _prop64(g1, src, dst)[:, :_N]
    g2 = _stage_b(t1, g1, dinv, b_e1.reshape(1, _HID), W_e2)
    t2 = _prop64(g2, src, dst)[:, :_N]
    g3 = _stage_c(t2, g2, dinv, b_e2.reshape(1, _HID), W_d1)
    t3 = _prop64(g3, src, dst)[:, :_N]
    g4 = _stage_d(t3, g3, dinv, b_d1.reshape(1, _HID))
    t4 = _prop64(g4, src, dst)[:, :_N]
    x_new = _stage_e(t4, g4, dinv, W_d2, b_d2.reshape(1, _DIM), x)
    t5a = _prop64(x_new[:, :_HID], src, dst)[:, :_N]
    t5b = _prop64(x_new[:, _HID:], src, dst)[:, :_N]
    lossm = _stage_f(x, x_new, t5a, t5b, dn)
    return (lossm[0, 0], x_new)


# speedup vs baseline: 27.6802x; 1.0781x over previous
import functools

import jax
import jax.numpy as jnp
from jax import lax
from jax.experimental import pallas as pl
from jax.experimental.pallas import tpu as pltpu
from jax.experimental.pallas import tpu_sc as plsc

_N = 10000
_E = 320000
_DIM = 128
_HID = 64
_ALPHA = 1.0
_BETA = 0.0
_GAMMA = 0.001

_NC = 2
_NS = 16
_NW = _NC * _NS
_EPT = _E // _NW
_CH = 80
_NCH = _EPT // _CH
_NPAD = 10240
_ZSEG = 640
_RPT = _NPAD // _NS
_ZROWS = 128

_BR = 1000
_GB = _N // _BR

_mesh = plsc.VectorSubcoreMesh(core_axis_name="c", subcore_axis_name="s")


@functools.partial(
    pl.kernel,
    out_type=jax.ShapeDtypeStruct((_NC * 2 * _NPAD,), jnp.float32),
    mesh=_mesh,
    compiler_params=pltpu.CompilerParams(use_tc_tiling_on_sc=False),
    scratch_types=[
        pltpu.VMEM((_NCH, _CH), jnp.int32),
        pltpu.VMEM((_NCH, _CH), jnp.int32),
        pltpu.VMEM((_CH,), jnp.float32),
        pltpu.VMEM((_ZSEG,), jnp.float32),
        pltpu.VMEM_SHARED((_NPAD,), jnp.float32),
        pltpu.VMEM_SHARED((_NPAD,), jnp.float32),
        pltpu.SemaphoreType.DMA,
    ],
)
def _deg_kernel(src_hbm, dst_hbm, out_hbm, src_v, dst_v, ones_v, zb_v,
                din_sh, dout_sh, sem):
    c = lax.axis_index("c")
    s = lax.axis_index("s")
    w = c * _NS + s
    one16 = jnp.full((16,), 1.0, jnp.float32)
    zero16 = jnp.zeros((16,), jnp.float32)
    for l in range(_CH // 16):
        ones_v[pl.ds(l * 16, 16)] = one16

    def _zf(i, _):
        zb_v[pl.ds(i * 16, 16)] = zero16
        return None

    lax.fori_loop(0, _ZSEG // 16, _zf, None)
    pltpu.sync_copy(zb_v, din_sh.at[pl.ds(s * _ZSEG, _ZSEG)])
    pltpu.sync_copy(zb_v, dout_sh.at[pl.ds(s * _ZSEG, _ZSEG)])
    pltpu.sync_copy(src_hbm.at[w], src_v)
    pltpu.sync_copy(dst_hbm.at[w], dst_v)
    plsc.subcore_barrier()

    def _body(j, _):
        pltpu.async_copy(ones_v, dout_sh.at[src_v.at[j]], sem, add=True)
        pltpu.async_copy(ones_v, din_sh.at[dst_v.at[j]], sem, add=True)

        @pl.when(j >= 8)
        def _drain():
            pltpu.make_async_copy(ones_v, dout_sh.at[src_v.at[0]], sem).wait()
            pltpu.make_async_copy(ones_v, din_sh.at[dst_v.at[0]], sem).wait()

        return None

    lax.fori_loop(0, _NCH, _body, None)

    def _dr(j, _):
        pltpu.make_async_copy(ones_v, dout_sh.at[src_v.at[0]], sem).wait()
        pltpu.make_async_copy(ones_v, din_sh.at[dst_v.at[0]], sem).wait()
        return None

    lax.fori_loop(0, 8, _dr, None)
    plsc.subcore_barrier()
    pltpu.sync_copy(din_sh.at[pl.ds(s * _ZSEG, _ZSEG)],
                    out_hbm.at[pl.ds(c * 2 * _NPAD + s * _ZSEG, _ZSEG)])
    pltpu.sync_copy(dout_sh.at[pl.ds(s * _ZSEG, _ZSEG)],
                    out_hbm.at[pl.ds(c * 2 * _NPAD + _NPAD + s * _ZSEG, _ZSEG)])


def _make_prop(d):
    @functools.partial(
        pl.kernel,
        out_type=jax.ShapeDtypeStruct((_NC, _NPAD, d), jnp.float32),
        mesh=_mesh,
        compiler_params=pltpu.CompilerParams(use_tc_tiling_on_sc=False),
        scratch_types=[
            pltpu.VMEM((_NCH, _CH), jnp.int32),
            pltpu.VMEM((_NCH, _CH), jnp.int32),
            pltpu.VMEM((_CH, d), jnp.float32),
            pltpu.VMEM((_CH, d), jnp.float32),
            pltpu.VMEM((_CH, d), jnp.float32),
            pltpu.VMEM((_CH, d), jnp.float32),
            pltpu.VMEM((_ZROWS, d), jnp.float32),
            pltpu.VMEM_SHARED((_NPAD, d), jnp.float32),
        ] + [pltpu.SemaphoreType.DMA] * 8,
    )
    def _prop(g_hbm, src_hbm, dst_hbm, out_hbm, src_v, dst_v, b0, b1, b2, b3,
              zb, acc, g0, g1, g2, g3, s0, s1, s2, s3):
        c = lax.axis_index("c")
        s = lax.axis_index("s")
        w = c * _NS + s
        bufs = (b0, b1, b2, b3)
        gsem = (g0, g1, g2, g3)
        ssem = (s0, s1, s2, s3)
        zero16 = jnp.zeros((16,), jnp.float32)

        def _zf(i, _):
            for l in range(d // 16):
                zb[i, pl.ds(l * 16, 16)] = zero16
            return None

        lax.fori_loop(0, _ZROWS, _zf, None)
        pltpu.sync_copy(src_hbm.at[w], src_v)
        pltpu.sync_copy(dst_hbm.at[w], dst_v)
        for z in range(_RPT // _ZROWS):
            pltpu.sync_copy(zb, acc.at[pl.ds(s * _RPT + z * _ZROWS, _ZROWS)])
        plsc.subcore_barrier()

        def _fire_g(j, b):
            pltpu.async_copy(g_hbm.at[src_v.at[j]], bufs[b], gsem[b])

        def _wait_g(j, b):
            pltpu.make_async_copy(g_hbm.at[src_v.at[j]], bufs[b], gsem[b]).wait()

        def _fire_s(j, b):
            pltpu.async_copy(bufs[b], acc.at[dst_v.at[j]], ssem[b], add=True)

        def _wait_s(j, b):
            pltpu.make_async_copy(bufs[b], acc.at[dst_v.at[j]], ssem[b]).wait()

        _fire_g(0, 0)
        _fire_g(1, 1)
        _fire_g(2, 2)
        _wait_g(0, 0)
        _fire_s(0, 0)
        _fire_g(3, 3)
        _wait_g(1, 1)
        _fire_s(1, 1)

        def _body(j2, _):
            jb = 2 + j2 * 4
            for u in range(4):
                j = jb + u
                _wait_s(j - 2, u)
                _fire_g(j + 2, u)
                _wait_g(j, (2 + u) % 4)
                _fire_s(j, (2 + u) % 4)
            return None

        lax.fori_loop(0, 30, _body, None)
        _wait_s(120, 0)
        _fire_g(124, 0)
        _wait_g(122, 2)
        _fire_s(122, 2)
        _wait_s(121, 1)
        _wait_g(123, 3)
        _fire_s(123, 3)
        _wait_s(122, 2)
        _wait_g(124, 0)
        _fire_s(124, 0)
        _wait_s(123, 3)
        _wait_s(124, 0)
        plsc.subcore_barrier()
        pltpu.sync_copy(acc.at[pl.ds(s * _RPT, _RPT)],
                        out_hbm.at[c, pl.ds(s * _RPT, _RPT)])

    return _prop


_prop64 = _make_prop(_HID)


def _stage_a_body(x_ref, dp_ref, w_ref, g1_ref, dinv_ref, dn_ref):
    dp = dp_ref[...]
    din = dp[:, 0:1] + dp[:, 2:3]
    dout = dp[:, 1:2] + dp[:, 3:4]
    dinv = lax.rsqrt(jnp.maximum(din + 1.0, 1e-12))
    dinv_ref[...] = dinv
    dn_ref[...] = din + dout
    g1_ref[...] = dinv * jnp.dot(x_ref[...], w_ref[...],
                                 preferred_element_type=jnp.float32)


_stage_a = pl.pallas_call(
    _stage_a_body,
    grid=(_GB,),
    in_specs=[
        pl.BlockSpec((_BR, _DIM), lambda i: (i, 0)),
        pl.BlockSpec((_BR, 4), lambda i: (i, 0)),
        pl.BlockSpec((_DIM, _HID), lambda i: (0, 0)),
    ],
    out_specs=[
        pl.BlockSpec((_BR, _HID), lambda i: (i, 0)),
        pl.BlockSpec((_BR, 1), lambda i: (i, 0)),
        pl.BlockSpec((_BR, 1), lambda i: (i, 0)),
    ],
    out_shape=[
        jax.ShapeDtypeStruct((_N, _HID), jnp.float32),
        jax.ShapeDtypeStruct((_N, 1), jnp.float32),
        jax.ShapeDtypeStruct((_N, 1), jnp.float32),
    ],
)


def _mid_body(relu, t_ref, g_ref, dinv_ref, b_ref, w_ref, out_ref):
    t = t_ref[0] + t_ref[1]
    pre = dinv_ref[...] * (t + g_ref[...]) + b_ref[...]
    h = jnp.maximum(pre, 0.0) if relu else pre
    out_ref[...] = dinv_ref[...] * jnp.dot(h, w_ref[...],
                                           preferred_element_type=jnp.float32)


def _make_mid(relu, dout):
    return pl.pallas_call(
        functools.partial(_mid_body, relu),
        grid=(_GB,),
        in_specs=[
            pl.BlockSpec((_NC, _BR, _HID), lambda i: (0, i, 0)),
            pl.BlockSpec((_BR, _HID), lambda i: (i, 0)),
            pl.BlockSpec((_BR, 1), lambda i: (i, 0)),
            pl.BlockSpec((1, _HID), lambda i: (0, 0)),
            pl.BlockSpec((_HID, dout), lambda i: (0, 0)),
        ],
        out_specs=pl.BlockSpec((_BR, dout), lambda i: (i, 0)),
        out_shape=jax.ShapeDtypeStruct((_N, dout), jnp.float32),
    )


_stage_b = _make_mid(True, _HID)
_stage_c = _make_mid(False, _HID)


def _stage_d_body(t_ref, g_ref, dinv_ref, b_ref, out_ref):
    t = t_ref[0] + t_ref[1]
    pre = dinv_ref[...] * (t + g_ref[...]) + b_ref[...]
    out_ref[...] = dinv_ref[...] * jnp.maximum(pre, 0.0)


_stage_d = pl.pallas_call(
    _stage_d_body,
    grid=(_GB,),
    in_specs=[
        pl.BlockSpec((_NC, _BR, _HID), lambda i: (0, i, 0)),
        pl.BlockSpec((_BR, _HID), lambda i: (i, 0)),
        pl.BlockSpec((_BR, 1), lambda i: (i, 0)),
        pl.BlockSpec((1, _HID), lambda i: (0, 0)),
    ],
    out_specs=pl.BlockSpec((_BR, _HID), lambda i: (i, 0)),
    out_shape=jax.ShapeDtypeStruct((_N, _HID), jnp.float32),
)


def _stage_e_body(t_ref, g_ref, dinv_ref, w_ref, b_ref, x_ref, out_ref,
                  xa_ref, xb_ref):
    t = t_ref[0] + t_ref[1]
    s4 = dinv_ref[...] * (t + g_ref[...])
    dec = jnp.dot(s4, w_ref[...], preferred_element_type=jnp.float32) + b_ref[...]
    xn = _BETA * x_ref[...] + (1.0 - _BETA) * dec
    out_ref[...] = xn
    xa_ref[...] = xn[:, :_HID]
    xb_ref[...] = xn[:, _HID:]


_stage_e = pl.pallas_call(
    _stage_e_body,
    grid=(_GB,),
    in_specs=[
        pl.BlockSpec((_NC, _BR, _HID), lambda i: (0, i, 0)),
        pl.BlockSpec((_BR, _HID), lambda i: (i, 0)),
        pl.BlockSpec((_BR, 1), lambda i: (i, 0)),
        pl.BlockSpec((_HID, _DIM), lambda i: (0, 0)),
        pl.BlockSpec((1, _DIM), lambda i: (0, 0)),
        pl.BlockSpec((_BR, _DIM), lambda i: (i, 0)),
    ],
    out_specs=[
        pl.BlockSpec((_BR, _DIM), lambda i: (i, 0)),
        pl.BlockSpec((_BR, _HID), lambda i: (i, 0)),
        pl.BlockSpec((_BR, _HID), lambda i: (i, 0)),
    ],
    out_shape=[
        jax.ShapeDtypeStruct((_N, _DIM), jnp.float32),
        jax.ShapeDtypeStruct((_N, _HID), jnp.float32),
        jax.ShapeDtypeStruct((_N, _HID), jnp.float32),
    ],
)


def _stage_f_body(x_ref, xn_ref, t5a_ref, t5b_ref, dn_ref, out_ref, acc):
    i = pl.program_id(0)

    @pl.when(i == 0)
    def _init():
        acc[0] = 0.0
        acc[1] = 0.0
        acc[2] = 0.0

    xn = xn_ref[...]
    t5a = t5a_ref[0] + t5a_ref[1]
    t5b = t5b_ref[0] + t5b_ref[1]
    cross = (jnp.sum(xn[:, :_HID] * t5a) + jnp.sum(xn[:, _HID:] * t5b))
    nrow = jnp.sum(xn * xn, axis=1, keepdims=True)
    esq = jnp.sum(nrow * dn_ref[...])
    dd = x_ref[...] - xn
    scs = jnp.sum(jnp.sqrt(jnp.sum(dd * dd, axis=1, keepdims=True) + 1e-12))
    acc[0] += cross
    acc[1] += esq
    acc[2] += scs

    @pl.when(i == _GB - 1)
    def _fin():
        out_ref[0, 0] = (_ALPHA * (acc[2] / _N)
                         + _GAMMA * ((acc[1] - 2.0 * acc[0]) / _E))


_stage_f = pl.pallas_call(
    _stage_f_body,
    grid=(_GB,),
    in_specs=[
        pl.BlockSpec((_BR, _DIM), lambda i: (i, 0)),
        pl.BlockSpec((_BR, _DIM), lambda i: (i, 0)),
        pl.BlockSpec((_NC, _BR, _HID), lambda i: (0, i, 0)),
        pl.BlockSpec((_NC, _BR, _HID), lambda i: (0, i, 0)),
        pl.BlockSpec((_BR, 1), lambda i: (i, 0)),
    ],
    out_specs=pl.BlockSpec(memory_space=pltpu.SMEM),
    out_shape=jax.ShapeDtypeStruct((1, 1), jnp.float32),
    scratch_shapes=[pltpu.SMEM((4,), jnp.float32)],
)


def kernel(x, f, s, W_e1, b_e1, W_e2, b_e2, W_d1, b_d1, W_d2, b_d2):
    src = s[0].reshape(_NW, _NCH, _CH)
    dst = s[1].reshape(_NW, _NCH, _CH)

    degp = _deg_kernel(src, dst).reshape(_NC, 2, _NPAD)
    degp2 = degp[:, :, :_N].transpose(2, 0, 1).reshape(_N, 4)

    g1, dinv, dn = _stage_a(x, degp2, W_e1)
    t1 = _prop64(g1, src, dst)
    g2 = _stage_b(t1, g1, dinv, b_e1.reshape(1, _HID), W_e2)
    t2 = _prop64(g2, src, dst)
    g3 = _stage_c(t2, g2, dinv, b_e2.reshape(1, _HID), W_d1)
    t3 = _prop64(g3, src, dst)
    g4 = _stage_d(t3, g3, dinv, b_d1.reshape(1, _HID))
    t4 = _prop64(g4, src, dst)
    x_new, xa, xb = _stage_e(t4, g4, dinv, W_d2, b_d2.reshape(1, _DIM), x)
    t5a = _prop64(xa, src, dst)
    t5b = _prop64(xb, src, dst)
    lossm = _stage_f(x, x_new, t5a, t5b, dn)
    return (lossm[0, 0], x_new)


# speedup vs baseline: 28.5352x; 1.0309x over previous
import functools

import jax
import jax.numpy as jnp
from jax import lax
from jax.experimental import pallas as pl
from jax.experimental.pallas import tpu as pltpu
from jax.experimental.pallas import tpu_sc as plsc

_N = 10000
_E = 320000
_DIM = 128
_HID = 64
_ALPHA = 1.0
_BETA = 0.0
_GAMMA = 0.001

_NC = 2
_NS = 16
_NW = _NC * _NS
_EPT = _E // _NW
_CH = 80
_NCH = _EPT // _CH
_NPAD = 10240
_ZSEG = 640
_RPT = _NPAD // _NS
_ZROWS = 128

_BR = 1000
_GB = _N // _BR

_mesh = plsc.VectorSubcoreMesh(core_axis_name="c", subcore_axis_name="s")


@functools.partial(
    pl.kernel,
    out_type=jax.ShapeDtypeStruct((_NC * 2 * _NPAD,), jnp.float32),
    mesh=_mesh,
    compiler_params=pltpu.CompilerParams(use_tc_tiling_on_sc=False),
    scratch_types=[
        pltpu.VMEM((_NCH, _CH), jnp.int32),
        pltpu.VMEM((_NCH, _CH), jnp.int32),
        pltpu.VMEM((_CH,), jnp.float32),
        pltpu.VMEM((_ZSEG,), jnp.float32),
        pltpu.VMEM_SHARED((_NPAD,), jnp.float32),
        pltpu.VMEM_SHARED((_NPAD,), jnp.float32),
        pltpu.SemaphoreType.DMA,
    ],
)
def _deg_kernel(src_hbm, dst_hbm, out_hbm, src_v, dst_v, ones_v, zb_v,
                din_sh, dout_sh, sem):
    c = lax.axis_index("c")
    s = lax.axis_index("s")
    w = c * _NS + s
    one16 = jnp.full((16,), 1.0, jnp.float32)
    zero16 = jnp.zeros((16,), jnp.float32)
    for l in range(_CH // 16):
        ones_v[pl.ds(l * 16, 16)] = one16

    def _zf(i, _):
        zb_v[pl.ds(i * 16, 16)] = zero16
        return None

    lax.fori_loop(0, _ZSEG // 16, _zf, None)
    pltpu.sync_copy(zb_v, din_sh.at[pl.ds(s * _ZSEG, _ZSEG)])
    pltpu.sync_copy(zb_v, dout_sh.at[pl.ds(s * _ZSEG, _ZSEG)])
    pltpu.sync_copy(src_hbm.at[w], src_v)
    pltpu.sync_copy(dst_hbm.at[w], dst_v)
    plsc.subcore_barrier()

    def _body(j, _):
        pltpu.async_copy(ones_v, dout_sh.at[src_v.at[j]], sem, add=True)
        pltpu.async_copy(ones_v, din_sh.at[dst_v.at[j]], sem, add=True)

        @pl.when(j >= 8)
        def _drain():
            pltpu.make_async_copy(ones_v, dout_sh.at[src_v.at[0]], sem).wait()
            pltpu.make_async_copy(ones_v, din_sh.at[dst_v.at[0]], sem).wait()

        return None

    lax.fori_loop(0, _NCH, _body, None)

    def _dr(j, _):
        pltpu.make_async_copy(ones_v, dout_sh.at[src_v.at[0]], sem).wait()
        pltpu.make_async_copy(ones_v, din_sh.at[dst_v.at[0]], sem).wait()
        return None

    lax.fori_loop(0, 8, _dr, None)
    plsc.subcore_barrier()
    pltpu.sync_copy(din_sh.at[pl.ds(s * _ZSEG, _ZSEG)],
                    out_hbm.at[pl.ds(c * 2 * _NPAD + s * _ZSEG, _ZSEG)])
    pltpu.sync_copy(dout_sh.at[pl.ds(s * _ZSEG, _ZSEG)],
                    out_hbm.at[pl.ds(c * 2 * _NPAD + _NPAD + s * _ZSEG, _ZSEG)])


def _make_prop(d):
    @functools.partial(
        pl.kernel,
        out_type=jax.ShapeDtypeStruct((_NC, _NPAD, d), jnp.float32),
        mesh=_mesh,
        compiler_params=pltpu.CompilerParams(use_tc_tiling_on_sc=False),
        scratch_types=[
            pltpu.VMEM((_NCH, _CH), jnp.int32),
            pltpu.VMEM((_NCH, _CH), jnp.int32),
        ] + [pltpu.VMEM((_CH, d), jnp.float32)] * 8 + [
            pltpu.VMEM((_ZROWS, d), jnp.float32),
            pltpu.VMEM_SHARED((_NPAD, d), jnp.float32),
        ] + [pltpu.SemaphoreType.DMA] * 16,
    )
    def _prop(g_hbm, src_hbm, dst_hbm, out_hbm, src_v, dst_v,
              b0, b1, b2, b3, b4, b5, b6, b7, zb, acc, *sems):
        c = lax.axis_index("c")
        s = lax.axis_index("s")
        w = c * _NS + s
        bufs = (b0, b1, b2, b3, b4, b5, b6, b7)
        gsem = sems[:8]
        ssem = sems[8:]
        zero16 = jnp.zeros((16,), jnp.float32)

        def _zf(i, _):
            for l in range(d // 16):
                zb[i, pl.ds(l * 16, 16)] = zero16
            return None

        lax.fori_loop(0, _ZROWS, _zf, None)
        pltpu.sync_copy(src_hbm.at[w], src_v)
        pltpu.sync_copy(dst_hbm.at[w], dst_v)
        for z in range(_RPT // _ZROWS):
            pltpu.sync_copy(zb, acc.at[pl.ds(s * _RPT + z * _ZROWS, _ZROWS)])
        plsc.subcore_barrier()

        def _fire_g(j, b):
            pltpu.async_copy(g_hbm.at[src_v.at[j]], bufs[b], gsem[b])

        def _wait_g(j, b):
            pltpu.make_async_copy(g_hbm.at[src_v.at[j]], bufs[b], gsem[b]).wait()

        def _fire_s(j, b):
            pltpu.async_copy(bufs[b], acc.at[dst_v.at[j]], ssem[b], add=True)

        def _wait_s(j, b):
            pltpu.make_async_copy(bufs[b], acc.at[dst_v.at[j]], ssem[b]).wait()

        def _step(j, jj=None):
            if jj is None:
                jj = j
            if j >= 4:
                _wait_s(jj - 4, (j - 4) % 8)
            if j + 4 <= _NCH - 1:
                _fire_g(jj + 4, (j + 4) % 8)
            _wait_g(jj, j % 8)
            _fire_s(jj, j % 8)

        for j in range(4):
            _fire_g(j, j)
        for j in range(4):
            _step(j)

        def _body(j2, _):
            jb = 4 + j2 * 8
            for u in range(8):
                _step(4 + u, jb + u)
            return None

        lax.fori_loop(0, 14, _body, None)
        for j in range(116, _NCH):
            _step(j)
        for j in range(_NCH - 4, _NCH):
            _wait_s(j, j % 8)
        plsc.subcore_barrier()
        pltpu.sync_copy(acc.at[pl.ds(s * _RPT, _RPT)],
                        out_hbm.at[c, pl.ds(s * _RPT, _RPT)])

    return _prop


_prop64 = _make_prop(_HID)


def _stage_a_body(x_ref, dp_ref, w_ref, g1_ref, dinv_ref, dn_ref):
    dp = dp_ref[...]
    din = dp[:, 0:1] + dp[:, 2:3]
    dout = dp[:, 1:2] + dp[:, 3:4]
    dinv = lax.rsqrt(jnp.maximum(din + 1.0, 1e-12))
    dinv_ref[...] = dinv
    dn_ref[...] = din + dout
    g1_ref[...] = dinv * jnp.dot(x_ref[...], w_ref[...],
                                 preferred_element_type=jnp.float32)


_stage_a = pl.pallas_call(
    _stage_a_body,
    grid=(_GB,),
    in_specs=[
        pl.BlockSpec((_BR, _DIM), lambda i: (i, 0)),
        pl.BlockSpec((_BR, 4), lambda i: (i, 0)),
        pl.BlockSpec((_DIM, _HID), lambda i: (0, 0)),
    ],
    out_specs=[
        pl.BlockSpec((_BR, _HID), lambda i: (i, 0)),
        pl.BlockSpec((_BR, 1), lambda i: (i, 0)),
        pl.BlockSpec((_BR, 1), lambda i: (i, 0)),
    ],
    out_shape=[
        jax.ShapeDtypeStruct((_N, _HID), jnp.float32),
        jax.ShapeDtypeStruct((_N, 1), jnp.float32),
        jax.ShapeDtypeStruct((_N, 1), jnp.float32),
    ],
)


def _mid_body(relu, t_ref, g_ref, dinv_ref, b_ref, w_ref, out_ref):
    t = t_ref[0] + t_ref[1]
    pre = dinv_ref[...] * (t + g_ref[...]) + b_ref[...]
    h = jnp.maximum(pre, 0.0) if relu else pre
    out_ref[...] = dinv_ref[...] * jnp.dot(h, w_ref[...],
                                           preferred_element_type=jnp.float32)


def _make_mid(relu, dout):
    return pl.pallas_call(
        functools.partial(_mid_body, relu),
        grid=(_GB,),
        in_specs=[
            pl.BlockSpec((_NC, _BR, _HID), lambda i: (0, i, 0)),
            pl.BlockSpec((_BR, _HID), lambda i: (i, 0)),
            pl.BlockSpec((_BR, 1), lambda i: (i, 0)),
            pl.BlockSpec((1, _HID), lambda i: (0, 0)),
            pl.BlockSpec((_HID, dout), lambda i: (0, 0)),
        ],
        out_specs=pl.BlockSpec((_BR, dout), lambda i: (i, 0)),
        out_shape=jax.ShapeDtypeStruct((_N, dout), jnp.float32),
    )


_stage_b = _make_mid(True, _HID)
_stage_c = _make_mid(False, _HID)


def _stage_d_body(t_ref, g_ref, dinv_ref, b_ref, out_ref):
    t = t_ref[0] + t_ref[1]
    pre = dinv_ref[...] * (t + g_ref[...]) + b_ref[...]
    out_ref[...] = dinv_ref[...] * jnp.maximum(pre, 0.0)


_stage_d = pl.pallas_call(
    _stage_d_body,
    grid=(_GB,),
    in_specs=[
        pl.BlockSpec((_NC, _BR, _HID), lambda i: (0, i, 0)),
        pl.BlockSpec((_BR, _HID), lambda i: (i, 0)),
        pl.BlockSpec((_BR, 1), lambda i: (i, 0)),
        pl.BlockSpec((1, _HID), lambda i: (0, 0)),
    ],
    out_specs=pl.BlockSpec((_BR, _HID), lambda i: (i, 0)),
    out_shape=jax.ShapeDtypeStruct((_N, _HID), jnp.float32),
)


def _stage_e_body(t_ref, g_ref, dinv_ref, w_ref, b_ref, x_ref, out_ref,
                  xa_ref, xb_ref):
    t = t_ref[0] + t_ref[1]
    s4 = dinv_ref[...] * (t + g_ref[...])
    dec = jnp.dot(s4, w_ref[...], preferred_element_type=jnp.float32) + b_ref[...]
    xn = _BETA * x_ref[...] + (1.0 - _BETA) * dec
    out_ref[...] = xn
    xa_ref[...] = xn[:, :_HID]
    xb_ref[...] = xn[:, _HID:]


_stage_e = pl.pallas_call(
    _stage_e_body,
    grid=(_GB,),
    in_specs=[
        pl.BlockSpec((_NC, _BR, _HID), lambda i: (0, i, 0)),
        pl.BlockSpec((_BR, _HID), lambda i: (i, 0)),
        pl.BlockSpec((_BR, 1), lambda i: (i, 0)),
        pl.BlockSpec((_HID, _DIM), lambda i: (0, 0)),
        pl.BlockSpec((1, _DIM), lambda i: (0, 0)),
        pl.BlockSpec((_BR, _DIM), lambda i: (i, 0)),
    ],
    out_specs=[
        pl.BlockSpec((_BR, _DIM), lambda i: (i, 0)),
        pl.BlockSpec((_BR, _HID), lambda i: (i, 0)),
        pl.BlockSpec((_BR, _HID), lambda i: (i, 0)),
    ],
    out_shape=[
        jax.ShapeDtypeStruct((_N, _DIM), jnp.float32),
        jax.ShapeDtypeStruct((_N, _HID), jnp.float32),
        jax.ShapeDtypeStruct((_N, _HID), jnp.float32),
    ],
)


def _stage_f_body(x_ref, xn_ref, t5a_ref, t5b_ref, dn_ref, out_ref, acc):
    i = pl.program_id(0)

    @pl.when(i == 0)
    def _init():
        acc[0] = 0.0
        acc[1] = 0.0
        acc[2] = 0.0

    xn = xn_ref[...]
    t5a = t5a_ref[0] + t5a_ref[1]
    t5b = t5b_ref[0] + t5b_ref[1]
    cross = (jnp.sum(xn[:, :_HID] * t5a) + jnp.sum(xn[:, _HID:] * t5b))
    nrow = jnp.sum(xn * xn, axis=1, keepdims=True)
    esq = jnp.sum(nrow * dn_ref[...])
    dd = x_ref[...] - xn
    scs = jnp.sum(jnp.sqrt(jnp.sum(dd * dd, axis=1, keepdims=True) + 1e-12))
    acc[0] += cross
    acc[1] += esq
    acc[2] += scs

    @pl.when(i == _GB - 1)
    def _fin():
        out_ref[0, 0] = (_ALPHA * (acc[2] / _N)
                         + _GAMMA * ((acc[1] - 2.0 * acc[0]) / _E))


_stage_f = pl.pallas_call(
    _stage_f_body,
    grid=(_GB,),
    in_specs=[
        pl.BlockSpec((_BR, _DIM), lambda i: (i, 0)),
        pl.BlockSpec((_BR, _DIM), lambda i: (i, 0)),
        pl.BlockSpec((_NC, _BR, _HID), lambda i: (0, i, 0)),
        pl.BlockSpec((_NC, _BR, _HID), lambda i: (0, i, 0)),
        pl.BlockSpec((_BR, 1), lambda i: (i, 0)),
    ],
    out_specs=pl.BlockSpec(memory_space=pltpu.SMEM),
    out_shape=jax.ShapeDtypeStruct((1, 1), jnp.float32),
    scratch_shapes=[pltpu.SMEM((4,), jnp.float32)],
)


def kernel(x, f, s, W_e1, b_e1, W_e2, b_e2, W_d1, b_d1, W_d2, b_d2):
    src = s[0].reshape(_NW, _NCH, _CH)
    dst = s[1].reshape(_NW, _NCH, _CH)

    degp = _deg_kernel(src, dst).reshape(_NC, 2, _NPAD)
    degp2 = degp[:, :, :_N].transpose(2, 0, 1).reshape(_N, 4)

    g1, dinv, dn = _stage_a(x, degp2, W_e1)
    t1 = _prop64(g1, src, dst)
    g2 = _stage_b(t1, g1, dinv, b_e1.reshape(1, _HID), W_e2)
    t2 = _prop64(g2, src, dst)
    g3 = _stage_c(t2, g2, dinv, b_e2.reshape(1, _HID), W_d1)
    t3 = _prop64(g3, src, dst)
    g4 = _stage_d(t3, g3, dinv, b_d1.reshape(1, _HID))
    t4 = _prop64(g4, src, dst)
    x_new, xa, xb = _stage_e(t4, g4, dinv, W_d2, b_d2.reshape(1, _DIM), x)
    t5a = _prop64(xa, src, dst)
    t5b = _prop64(xb, src, dst)
    lossm = _stage_f(x, x_new, t5a, t5b, dn)
    return (lossm[0, 0], x_new)


# speedup vs baseline: 32.0367x; 1.1227x over previous
import functools

import jax
import jax.numpy as jnp
from jax import lax
from jax.experimental import pallas as pl
from jax.experimental.pallas import tpu as pltpu
from jax.experimental.pallas import tpu_sc as plsc

_N = 10000
_E = 320000
_DIM = 128
_HID = 64
_ALPHA = 1.0
_BETA = 0.0
_GAMMA = 0.001

_NC = 2
_NS = 16
_NW = _NC * _NS
_EPT = _E // _NW
_CH = 80
_NCH = _EPT // _CH
_NPAD = 10240
_ZSEG = 640
_RPT = _NPAD // _NS
_ZROWS = 128

_BR = 1000
_GB = _N // _BR

_mesh = plsc.VectorSubcoreMesh(core_axis_name="c", subcore_axis_name="s")


@functools.partial(
    pl.kernel,
    out_type=jax.ShapeDtypeStruct((_NC * 2 * _NPAD,), jnp.float32),
    mesh=_mesh,
    compiler_params=pltpu.CompilerParams(use_tc_tiling_on_sc=False),
    scratch_types=[
        pltpu.VMEM((_NCH, _CH), jnp.int32),
        pltpu.VMEM((_NCH, _CH), jnp.int32),
        pltpu.VMEM((_CH,), jnp.float32),
        pltpu.VMEM((_ZSEG,), jnp.float32),
        pltpu.VMEM_SHARED((_NPAD,), jnp.float32),
        pltpu.VMEM_SHARED((_NPAD,), jnp.float32),
        pltpu.SemaphoreType.DMA,
    ],
)
def _deg_kernel(src_hbm, dst_hbm, out_hbm, src_v, dst_v, ones_v, zb_v,
                din_sh, dout_sh, sem):
    c = lax.axis_index("c")
    s = lax.axis_index("s")
    w = c * _NS + s
    one16 = jnp.full((16,), 1.0, jnp.float32)
    zero16 = jnp.zeros((16,), jnp.float32)
    for l in range(_CH // 16):
        ones_v[pl.ds(l * 16, 16)] = one16

    def _zf(i, _):
        zb_v[pl.ds(i * 16, 16)] = zero16
        return None

    lax.fori_loop(0, _ZSEG // 16, _zf, None)
    pltpu.sync_copy(zb_v, din_sh.at[pl.ds(s * _ZSEG, _ZSEG)])
    pltpu.sync_copy(zb_v, dout_sh.at[pl.ds(s * _ZSEG, _ZSEG)])
    pltpu.sync_copy(src_hbm.at[w], src_v)
    pltpu.sync_copy(dst_hbm.at[w], dst_v)
    plsc.subcore_barrier()

    def _body(j, _):
        pltpu.async_copy(ones_v, dout_sh.at[src_v.at[j]], sem, add=True)
        pltpu.async_copy(ones_v, din_sh.at[dst_v.at[j]], sem, add=True)

        @pl.when(j >= 8)
        def _drain():
            pltpu.make_async_copy(ones_v, dout_sh.at[src_v.at[0]], sem).wait()
            pltpu.make_async_copy(ones_v, din_sh.at[dst_v.at[0]], sem).wait()

        return None

    lax.fori_loop(0, _NCH, _body, None)

    def _dr(j, _):
        pltpu.make_async_copy(ones_v, dout_sh.at[src_v.at[0]], sem).wait()
        pltpu.make_async_copy(ones_v, din_sh.at[dst_v.at[0]], sem).wait()
        return None

    lax.fori_loop(0, 8, _dr, None)
    plsc.subcore_barrier()
    pltpu.sync_copy(din_sh.at[pl.ds(s * _ZSEG, _ZSEG)],
                    out_hbm.at[pl.ds(c * 2 * _NPAD + s * _ZSEG, _ZSEG)])
    pltpu.sync_copy(dout_sh.at[pl.ds(s * _ZSEG, _ZSEG)],
                    out_hbm.at[pl.ds(c * 2 * _NPAD + _NPAD + s * _ZSEG, _ZSEG)])


def _make_prop(d):
    @functools.partial(
        pl.kernel,
        out_type=jax.ShapeDtypeStruct((_NC, _NPAD, d), jnp.float32),
        mesh=_mesh,
        compiler_params=pltpu.CompilerParams(use_tc_tiling_on_sc=False),
        scratch_types=[
            pltpu.VMEM((_NCH, _CH), jnp.int32),
            pltpu.VMEM((_NCH, _CH), jnp.int32),
        ] + [pltpu.VMEM((_CH, d), jnp.float32)] * 8 + [
            pltpu.VMEM((_ZROWS, d), jnp.float32),
            pltpu.VMEM_SHARED((_NPAD, d), jnp.float32),
        ] + [pltpu.SemaphoreType.DMA] * 16,
    )
    def _prop(g_hbm, src_hbm, dst_hbm, out_hbm, src_v, dst_v,
              b0, b1, b2, b3, b4, b5, b6, b7, zb, acc, *sems):
        c = lax.axis_index("c")
        s = lax.axis_index("s")
        w = c * _NS + s
        bufs = (b0, b1, b2, b3, b4, b5, b6, b7)
        gsem = sems[:8]
        ssem = sems[8:]
        zero16 = jnp.zeros((16,), jnp.float32)

        def _zf(i, _):
            for l in range(d // 16):
                zb[i, pl.ds(l * 16, 16)] = zero16
            return None

        lax.fori_loop(0, _ZROWS, _zf, None)
        pltpu.sync_copy(src_hbm.at[w], src_v)
        pltpu.sync_copy(dst_hbm.at[w], dst_v)
        for z in range(_RPT // _ZROWS):
            pltpu.sync_copy(zb, acc.at[pl.ds(s * _RPT + z * _ZROWS, _ZROWS)])
        plsc.subcore_barrier()

        def _fire_g(j, b):
            pltpu.async_copy(g_hbm.at[src_v.at[j]], bufs[b], gsem[b])

        def _wait_g(j, b):
            pltpu.make_async_copy(g_hbm.at[src_v.at[j]], bufs[b], gsem[b]).wait()

        def _fire_s(j, b):
            pltpu.async_copy(bufs[b], acc.at[dst_v.at[j]], ssem[b], add=True)

        def _wait_s(j, b):
            pltpu.make_async_copy(bufs[b], acc.at[dst_v.at[j]], ssem[b]).wait()

        def _step(j, jj=None):
            if jj is None:
                jj = j
            if j >= 4:
                _wait_s(jj - 4, (j - 4) % 8)
            if j + 4 <= _NCH - 1:
                _fire_g(jj + 4, (j + 4) % 8)
            _wait_g(jj, j % 8)
            _fire_s(jj, j % 8)

        for j in range(4):
            _fire_g(j, j)
        for j in range(4):
            _step(j)

        def _body(j2, _):
            jb = 4 + j2 * 8
            for u in range(8):
                _step(4 + u, jb + u)
            return None

        lax.fori_loop(0, 14, _body, None)
        for j in range(116, _NCH):
            _step(j)
        for j in range(_NCH - 4, _NCH):
            _wait_s(j, j % 8)
        plsc.subcore_barrier()
        pltpu.sync_copy(acc.at[pl.ds(s * _RPT, _RPT)],
                        out_hbm.at[c, pl.ds(s * _RPT, _RPT)])

    return _prop


_prop64 = _make_prop(_HID)


def _stage_a_body(x_ref, dp_ref, w_ref, g1_ref, dinv_ref, dn_ref):
    dp = dp_ref[...]
    din = dp[:, 0:1] + dp[:, 2:3]
    dout = dp[:, 1:2] + dp[:, 3:4]
    dinv = lax.rsqrt(jnp.maximum(din + 1.0, 1e-12))
    dinv_ref[...] = dinv
    dn_ref[...] = din + dout
    g1_ref[...] = dinv * jnp.dot(x_ref[...], w_ref[...],
                                 preferred_element_type=jnp.float32)


_stage_a = pl.pallas_call(
    _stage_a_body,
    grid=(_GB,),
    in_specs=[
        pl.BlockSpec((_BR, _DIM), lambda i: (i, 0)),
        pl.BlockSpec((_BR, 4), lambda i: (i, 0)),
        pl.BlockSpec((_DIM, _HID), lambda i: (0, 0)),
    ],
    out_specs=[
        pl.BlockSpec((_BR, _HID), lambda i: (i, 0)),
        pl.BlockSpec((_BR, 1), lambda i: (i, 0)),
        pl.BlockSpec((_BR, 1), lambda i: (i, 0)),
    ],
    out_shape=[
        jax.ShapeDtypeStruct((_N, _HID), jnp.float32),
        jax.ShapeDtypeStruct((_N, 1), jnp.float32),
        jax.ShapeDtypeStruct((_N, 1), jnp.float32),
    ],
)


def _mid_body(relu, t_ref, g_ref, dinv_ref, b_ref, w_ref, out_ref):
    t = t_ref[0] + t_ref[1]
    pre = dinv_ref[...] * (t + g_ref[...]) + b_ref[...]
    h = jnp.maximum(pre, 0.0) if relu else pre
    out_ref[...] = dinv_ref[...] * jnp.dot(h, w_ref[...],
                                           preferred_element_type=jnp.float32)


def _make_mid(relu, dout):
    return pl.pallas_call(
        functools.partial(_mid_body, relu),
        grid=(_GB,),
        in_specs=[
            pl.BlockSpec((_NC, _BR, _HID), lambda i: (0, i, 0)),
            pl.BlockSpec((_BR, _HID), lambda i: (i, 0)),
            pl.BlockSpec((_BR, 1), lambda i: (i, 0)),
            pl.BlockSpec((1, _HID), lambda i: (0, 0)),
            pl.BlockSpec((_HID, dout), lambda i: (0, 0)),
        ],
        out_specs=pl.BlockSpec((_BR, dout), lambda i: (i, 0)),
        out_shape=jax.ShapeDtypeStruct((_N, dout), jnp.float32),
    )


_stage_b = _make_mid(True, _HID)
_stage_c = _make_mid(False, _HID)


def _stage_d_body(t_ref, g_ref, dinv_ref, b_ref, out_ref):
    t = t_ref[0] + t_ref[1]
    pre = dinv_ref[...] * (t + g_ref[...]) + b_ref[...]
    out_ref[...] = dinv_ref[...] * jnp.maximum(pre, 0.0)


_stage_d = pl.pallas_call(
    _stage_d_body,
    grid=(_GB,),
    in_specs=[
        pl.BlockSpec((_NC, _BR, _HID), lambda i: (0, i, 0)),
        pl.BlockSpec((_BR, _HID), lambda i: (i, 0)),
        pl.BlockSpec((_BR, 1), lambda i: (i, 0)),
        pl.BlockSpec((1, _HID), lambda i: (0, 0)),
    ],
    out_specs=pl.BlockSpec((_BR, _HID), lambda i: (i, 0)),
    out_shape=jax.ShapeDtypeStruct((_N, _HID), jnp.float32),
)


def _stage_e_body(t_ref, g_ref, dinv_ref, w_ref, b_ref, x_ref, out_ref,
                  s4_ref, s4m_ref):
    t = t_ref[0] + t_ref[1]
    s4 = dinv_ref[...] * (t + g_ref[...])
    w = w_ref[...]
    s4w = jnp.dot(s4, w, preferred_element_type=jnp.float32)
    dec = s4w + b_ref[...]
    out_ref[...] = _BETA * x_ref[...] + (1.0 - _BETA) * dec
    s4_ref[...] = s4
    s4m_ref[...] = jnp.dot(s4w, w.T, preferred_element_type=jnp.float32)


_stage_e = pl.pallas_call(
    _stage_e_body,
    grid=(_GB,),
    in_specs=[
        pl.BlockSpec((_NC, _BR, _HID), lambda i: (0, i, 0)),
        pl.BlockSpec((_BR, _HID), lambda i: (i, 0)),
        pl.BlockSpec((_BR, 1), lambda i: (i, 0)),
        pl.BlockSpec((_HID, _DIM), lambda i: (0, 0)),
        pl.BlockSpec((1, _DIM), lambda i: (0, 0)),
        pl.BlockSpec((_BR, _DIM), lambda i: (i, 0)),
    ],
    out_specs=[
        pl.BlockSpec((_BR, _DIM), lambda i: (i, 0)),
        pl.BlockSpec((_BR, _HID), lambda i: (i, 0)),
        pl.BlockSpec((_BR, _HID), lambda i: (i, 0)),
    ],
    out_shape=[
        jax.ShapeDtypeStruct((_N, _DIM), jnp.float32),
        jax.ShapeDtypeStruct((_N, _HID), jnp.float32),
        jax.ShapeDtypeStruct((_N, _HID), jnp.float32),
    ],
)


def _stage_f_body(x_ref, xn_ref, t5_ref, s4_ref, s4m_ref, dn_ref, w_ref,
                  b_ref, out_ref, acc, vec):
    i = pl.program_id(0)

    @pl.when(i == 0)
    def _init():
        acc[0] = 0.0
        acc[1] = 0.0
        acc[2] = 0.0
        vec[...] = jnp.zeros((1, _HID), jnp.float32)

    xn = xn_ref[...]
    dn = dn_ref[...]
    t5 = t5_ref[0] + t5_ref[1]
    cross = jnp.sum(t5 * s4m_ref[...])
    nrow = jnp.sum(xn * xn, axis=1, keepdims=True)
    esq = jnp.sum(nrow * dn)
    dd = x_ref[...] - xn
    scs = jnp.sum(jnp.sqrt(jnp.sum(dd * dd, axis=1, keepdims=True) + 1e-12))
    acc[0] += cross
    acc[1] += esq
    acc[2] += scs
    vec[...] += jnp.sum(dn * s4_ref[...], axis=0, keepdims=True)

    @pl.when(i == _GB - 1)
    def _fin():
        b = b_ref[...]
        wb = jnp.sum(w_ref[...] * b, axis=1, keepdims=True)
        cross_tot = (acc[0] + jnp.sum(vec[...] * wb.T)
                     + _E * jnp.sum(b * b))
        out_ref[0, 0] = (_ALPHA * (acc[2] / _N)
                         + _GAMMA * ((acc[1] - 2.0 * cross_tot) / _E))


_stage_f = pl.pallas_call(
    _stage_f_body,
    grid=(_GB,),
    in_specs=[
        pl.BlockSpec((_BR, _DIM), lambda i: (i, 0)),
        pl.BlockSpec((_BR, _DIM), lambda i: (i, 0)),
        pl.BlockSpec((_NC, _BR, _HID), lambda i: (0, i, 0)),
        pl.BlockSpec((_BR, _HID), lambda i: (i, 0)),
        pl.BlockSpec((_BR, _HID), lambda i: (i, 0)),
        pl.BlockSpec((_BR, 1), lambda i: (i, 0)),
        pl.BlockSpec((_HID, _DIM), lambda i: (0, 0)),
        pl.BlockSpec((1, _DIM), lambda i: (0, 0)),
    ],
    out_specs=pl.BlockSpec(memory_space=pltpu.SMEM),
    out_shape=jax.ShapeDtypeStruct((1, 1), jnp.float32),
    scratch_shapes=[pltpu.SMEM((4,), jnp.float32),
                    pltpu.VMEM((1, _HID), jnp.float32)],
)


def kernel(x, f, s, W_e1, b_e1, W_e2, b_e2, W_d1, b_d1, W_d2, b_d2):
    src = s[0].reshape(_NW, _NCH, _CH)
    dst = s[1].reshape(_NW, _NCH, _CH)

    degp = _deg_kernel(src, dst).reshape(_NC, 2, _NPAD)
    degp2 = degp[:, :, :_N].transpose(2, 0, 1).reshape(_N, 4)

    g1, dinv, dn = _stage_a(x, degp2, W_e1)
    t1 = _prop64(g1, src, dst)
    g2 = _stage_b(t1, g1, dinv, b_e1.reshape(1, _HID), W_e2)
    t2 = _prop64(g2, src, dst)
    g3 = _stage_c(t2, g2, dinv, b_e2.reshape(1, _HID), W_d1)
    t3 = _prop64(g3, src, dst)
    g4 = _stage_d(t3, g3, dinv, b_d1.reshape(1, _HID))
    t4 = _prop64(g4, src, dst)
    x_new, s4, s4m = _stage_e(t4, g4, dinv, W_d2, b_d2.reshape(1, _DIM), x)
    t5 = _prop64(s4, src, dst)
    lossm = _stage_f(x, x_new, t5, s4, s4m, dn, W_d2, b_d2.reshape(1, _DIM))
    return (lossm[0, 0], x_new)
